# baseline (speedup 1.0000x reference)
import jax
import jax.numpy as jnp
from jax.experimental import pallas as pl
from jax.experimental.pallas import tpu as pltpu

_OH = 7
_OW = 7


def _roi_body(bounds_ref, feat_ref, out_ref, tx_ref):
    pb = pl.program_id(0)
    pn = pl.program_id(1)
    h = feat_ref.shape[2]

    @pl.when(pn == 0)
    def _build():
        t0 = feat_ref[0]
        t1 = jnp.maximum(t0, jnp.concatenate([t0[1:], t0[-1:]], axis=0))
        t2 = jnp.maximum(t1, jnp.concatenate([t1[2:], t1[-2:]], axis=0))
        tx_ref[0] = t0
        tx_ref[1] = t1
        tx_ref[2] = t2

    cms = []
    for jj in range(_OW):
        xs = bounds_ref[pb, pn, jj]
        xb = bounds_ref[pb, pn, _OW + jj]
        kx = bounds_ref[pb, pn, 2 * _OW + jj]
        cms.append(jnp.maximum(tx_ref[kx, xs], tx_ref[kx, xb]))
    cmall = jnp.concatenate(cms, axis=-1)

    neg = jnp.array(-jnp.inf, dtype=cmall.dtype)
    zero = jnp.array(0.0, dtype=cmall.dtype)
    ridx = jax.lax.broadcasted_iota(jnp.int32, (h, 1), 0)
    vflag = bounds_ref[pb, pn, 5 * _OW]
    rows = []
    for ii in range(_OH):
        ys = bounds_ref[pb, pn, 3 * _OW + ii]
        ye = bounds_ref[pb, pn, 4 * _OW + ii]
        rm = (ridx >= ys) & (ridx < ye)
        rows.append(jnp.max(jnp.where(rm, cmall, neg), axis=0, keepdims=True))
    res = jnp.concatenate(rows, axis=0)
    res = jnp.where(vflag > 0, res, zero)
    cells = res.reshape(_OH * _OW, -1)
    out_ref[0, 0] = cells.T


def kernel(features, rois):
    b, c, h, w = features.shape
    n = rois.shape[1]

    x1 = jnp.maximum(0, (rois[..., 0] * w).astype(jnp.int32))
    y1 = jnp.maximum(0, (rois[..., 1] * h).astype(jnp.int32))
    x2 = jnp.minimum(w - 1, (rois[..., 2] * w).astype(jnp.int32))
    y2 = jnp.minimum(h - 1, (rois[..., 3] * h).astype(jnp.int32))
    valid = (x2 >= x1) & (y2 >= y1)
    rw = x2 - x1 + 1
    rh = y2 - y1 + 1
    jj = jnp.arange(_OW)
    ii = jnp.arange(_OH)
    xs = x1[..., None] + (jj * rw[..., None]) // _OW
    xe = x1[..., None] + -((-(jj + 1) * rw[..., None]) // _OW)
    ys = y1[..., None] + (ii * rh[..., None]) // _OH
    ye = y1[..., None] + -((-(ii + 1) * rh[..., None]) // _OH)
    lenx = jnp.maximum(xe - xs, 1)
    kx = (lenx >= 2).astype(jnp.int32) + (lenx >= 4).astype(jnp.int32)
    xb = xe - jnp.left_shift(1, kx)
    xs_c = jnp.clip(xs, 0, w - 1)
    xb_c = jnp.clip(xb, 0, w - 1)
    bounds = jnp.concatenate(
        [xs_c, xb_c, kx, ys, ye, valid[..., None].astype(jnp.int32)], axis=-1
    )

    feat_t = features.transpose(0, 3, 2, 1)

    out = pl.pallas_call(
        _roi_body,
        grid_spec=pltpu.PrefetchScalarGridSpec(
            num_scalar_prefetch=1,
            grid=(b, n),
            in_specs=[
                pl.BlockSpec((1, w, h, c), lambda pb, pn, bnds: (pb, 0, 0, 0)),
            ],
            out_specs=pl.BlockSpec(
                (1, 1, c, _OH * _OW), lambda pb, pn, bnds: (pb, pn, 0, 0)
            ),
            scratch_shapes=[pltpu.VMEM((3, w, h, c), features.dtype)],
        ),
        out_shape=jax.ShapeDtypeStruct((b, n, c, _OH * _OW), features.dtype),
    )(bounds, feat_t)

    return out.reshape(b, n, c, _OH, _OW)
